# baseline (device time: 153810 ns/iter reference)
import jax
import jax.numpy as jnp
from jax import lax
from jax.experimental import pallas as pl
from jax.experimental.pallas import tpu as pltpu

B, Sq, D, Hq, Dh = 4, 256, 1024, 8, 128
SCALE = 0.08838834764831843


def kernel(x, Wq, Wo, K_ext, V_ext):
    xb = x.astype(jnp.bfloat16)
    wq = Wq.astype(jnp.bfloat16)
    wo = Wo.astype(jnp.bfloat16)
    kt = K_ext
    vt = V_ext

    def body(x_ref, wq_ref, wo_ref, k_ref, v_ref, out_ref,
             q_scr, o_scr, ml_scr, a_scr, o_recv, ml_recv,
             send_sems, recv_sems):
        o_send = q_scr
        my = lax.axis_index("i")
        p1 = my ^ 1
        p2 = 3 - my

        bar = pltpu.get_barrier_semaphore()
        for nbr in (p1, p2):
            pl.semaphore_signal(bar, inc=1, device_id=(nbr,),
                                device_id_type=pl.DeviceIdType.MESH)
        pl.semaphore_wait(bar, 2)

        def attn_body(b, c):
            qb = lax.dot_general(x_ref[b], wq_ref[...], (((1,), (0,)), ((), ())),
                                 preferred_element_type=jnp.float32)
            q_scr[b] = (qb * SCALE).astype(jnp.bfloat16)
            for h in range(Hq):
                cols = slice(h * Dh, (h + 1) * Dh)
                qh = q_scr[b, :, cols]
                kh = k_ref[b, :, h, :].astype(jnp.bfloat16)
                vh = v_ref[b, :, h, :].astype(jnp.bfloat16)
                s = lax.dot_general(qh, kh, (((1,), (1,)), ((), ())),
                                    preferred_element_type=jnp.float32)
                m = jnp.max(s, axis=1, keepdims=True)
                p = jnp.exp(s - m)
                l = jnp.sum(p, axis=1, keepdims=True)
                o = lax.dot_general(p.astype(jnp.bfloat16), vh,
                                    (((1,), (0,)), ((), ())),
                                    preferred_element_type=jnp.float32)
                o_scr[b, :, cols] = o
                ml_scr[b, :, h:h + 1] = m
                ml_scr[b, :, Hq + h:Hq + h + 1] = l
            return c
        lax.fori_loop(0, B, attn_body, 0)

        for stage in range(2):
            partner = p1 if stage == 0 else p2
            o_send[...] = o_scr[...].astype(jnp.bfloat16)
            rdma_o = pltpu.make_async_remote_copy(
                src_ref=o_send, dst_ref=o_recv.at[stage],
                send_sem=send_sems.at[2 * stage], recv_sem=recv_sems.at[2 * stage],
                device_id=(partner,), device_id_type=pl.DeviceIdType.MESH)
            rdma_ml = pltpu.make_async_remote_copy(
                src_ref=ml_scr, dst_ref=ml_recv.at[stage],
                send_sem=send_sems.at[2 * stage + 1],
                recv_sem=recv_sems.at[2 * stage + 1],
                device_id=(partner,), device_id_type=pl.DeviceIdType.MESH)
            rdma_o.start()
            rdma_ml.start()
            rdma_o.wait()
            rdma_ml.wait()

            ml_a = ml_scr[...]
            ml_b = ml_recv[stage]
            m_new = jnp.maximum(ml_a[:, :, :Hq], ml_b[:, :, :Hq])
            a_a = jnp.exp(ml_a[:, :, :Hq] - m_new)
            a_b = jnp.exp(ml_b[:, :, :Hq] - m_new)
            ml_scr[:, :, :Hq] = m_new
            ml_scr[:, :, Hq:] = ml_a[:, :, Hq:] * a_a + ml_b[:, :, Hq:] * a_b
            a_scr[:, :, :Hq] = a_a
            a_scr[:, :, Hq:] = a_b

            def merge_body(b, c, stage=stage):
                for h in range(Hq):
                    cols = slice(h * Dh, (h + 1) * Dh)
                    o_scr[b, :, cols] = (
                        o_scr[b, :, cols] * a_scr[b, :, h:h + 1]
                        + o_recv[stage, b, :, cols].astype(jnp.float32)
                        * a_scr[b, :, Hq + h:Hq + h + 1])
                return c
            lax.fori_loop(0, B, merge_body, 0)

        a_scr[:, :, :Hq] = 1.0 / ml_scr[:, :, Hq:]

        def out_body(b, c):
            for h in range(Hq):
                cols = slice(h * Dh, (h + 1) * Dh)
                o_scr[b, :, cols] = o_scr[b, :, cols] * a_scr[b, :, h:h + 1]
            out_ref[b] = lax.dot_general(o_scr[b].astype(jnp.bfloat16), wo_ref[...],
                                         (((1,), (0,)), ((), ())),
                                         preferred_element_type=jnp.float32)
            return c
        lax.fori_loop(0, B, out_body, 0)

    return pl.pallas_call(
        body,
        out_shape=jax.ShapeDtypeStruct((B, Sq, D), jnp.float32),
        in_specs=[pl.BlockSpec(memory_space=pltpu.VMEM)] * 5,
        out_specs=pl.BlockSpec(memory_space=pltpu.VMEM),
        scratch_shapes=[
            pltpu.VMEM((B, Sq, D), jnp.bfloat16),
            pltpu.VMEM((B, Sq, D), jnp.float32),
            pltpu.VMEM((B, Sq, 2 * Hq), jnp.float32),
            pltpu.VMEM((B, Sq, 2 * Hq), jnp.float32),
            pltpu.VMEM((2, B, Sq, D), jnp.bfloat16),
            pltpu.VMEM((2, B, Sq, 2 * Hq), jnp.float32),
            pltpu.SemaphoreType.DMA((4,)),
            pltpu.SemaphoreType.DMA((4,)),
        ],
        compiler_params=pltpu.CompilerParams(
            collective_id=0, vmem_limit_bytes=100 * 1024 * 1024),
    )(xb, wq, wo, kt, vt)


# device time: 92082 ns/iter; 1.6704x vs baseline; 1.6704x over previous
import jax
import jax.numpy as jnp
from jax import lax
from jax.experimental import pallas as pl
from jax.experimental.pallas import tpu as pltpu

B, Sq, D, Hq, Dh = 4, 256, 1024, 8, 128
NBH = B * Hq
SCALE = 0.08838834764831843


def kernel(x, Wq, Wo, K_ext, V_ext):
    Skv = K_ext.shape[1]
    xb = x.astype(jnp.bfloat16)
    wq = Wq.astype(jnp.bfloat16)
    wo = Wo.astype(jnp.bfloat16)

    def body(x_ref, wq_ref, wo_ref, k_ref, v_ref, out_ref,
             q_scr, o_scr, ml_scr, a_scr, o_recv, ml_recv,
             k_buf, v_buf, kv_sems, send_sems, recv_sems):
        o_send = q_scr
        my = lax.axis_index("i")
        p1 = my ^ 1
        p2 = 3 - my

        bar = pltpu.get_barrier_semaphore()
        for nbr in (p1, p2):
            pl.semaphore_signal(bar, inc=1, device_id=(nbr,),
                                device_id_type=pl.DeviceIdType.MESH)
        pl.semaphore_wait(bar, 2)

        def kv_copies(i, slot):
            b = i // Hq
            h = i % Hq
            ck = pltpu.make_async_copy(
                k_ref.at[b, :, h, :], k_buf.at[slot], kv_sems.at[slot, 0])
            cv = pltpu.make_async_copy(
                v_ref.at[b, :, h, :], v_buf.at[slot], kv_sems.at[slot, 1])
            return ck, cv

        ck0, cv0 = kv_copies(0, 0)
        ck0.start()
        cv0.start()

        def attn_body(b, c):
            qb = lax.dot_general(x_ref[b], wq_ref[...], (((1,), (0,)), ((), ())),
                                 preferred_element_type=jnp.float32)
            q_scr[b] = (qb * SCALE).astype(jnp.bfloat16)
            for h in range(Hq):
                i = b * Hq + h
                slot = h % 2

                @pl.when(i + 1 < NBH)
                def _():
                    ck, cv = kv_copies(i + 1, (h + 1) % 2)
                    ck.start()
                    cv.start()

                ck, cv = kv_copies(i, slot)
                ck.wait()
                cv.wait()
                cols = slice(h * Dh, (h + 1) * Dh)
                qh = q_scr[b, :, cols]
                kh = k_buf[slot].astype(jnp.bfloat16)
                vh = v_buf[slot].astype(jnp.bfloat16)
                s = lax.dot_general(qh, kh, (((1,), (1,)), ((), ())),
                                    preferred_element_type=jnp.float32)
                m = jnp.max(s, axis=1, keepdims=True)
                p = jnp.exp(s - m)
                l = jnp.sum(p, axis=1, keepdims=True)
                o = lax.dot_general(p.astype(jnp.bfloat16), vh,
                                    (((1,), (0,)), ((), ())),
                                    preferred_element_type=jnp.float32)
                o_scr[b, :, cols] = o
                ml_scr[b, :, h:h + 1] = m
                ml_scr[b, :, Hq + h:Hq + h + 1] = l
            return c

        def exchange(stage, half, partner):
            rows = pl.ds(2 * half, 2)
            idx = stage * 4 + half * 2
            r_o = pltpu.make_async_remote_copy(
                src_ref=o_send.at[rows], dst_ref=o_recv.at[stage, rows],
                send_sem=send_sems.at[idx], recv_sem=recv_sems.at[idx],
                device_id=(partner,), device_id_type=pl.DeviceIdType.MESH)
            r_ml = pltpu.make_async_remote_copy(
                src_ref=ml_scr.at[rows], dst_ref=ml_recv.at[stage, rows],
                send_sem=send_sems.at[idx + 1], recv_sem=recv_sems.at[idx + 1],
                device_id=(partner,), device_id_type=pl.DeviceIdType.MESH)
            return r_o, r_ml

        def cast_half(half):
            rows = pl.ds(2 * half, 2)
            o_send[rows] = o_scr[rows].astype(jnp.bfloat16)

        def merge(stage, half):
            rows = slice(2 * half, 2 * half + 2)
            ml_a = ml_scr[rows]
            ml_b = ml_recv[stage, rows]
            m_new = jnp.maximum(ml_a[..., :Hq], ml_b[..., :Hq])
            a_a = jnp.exp(ml_a[..., :Hq] - m_new)
            a_b = jnp.exp(ml_b[..., :Hq] - m_new)
            ml_scr[rows, :, :Hq] = m_new
            ml_scr[rows, :, Hq:] = ml_a[..., Hq:] * a_a + ml_b[..., Hq:] * a_b
            a_scr[rows, :, :Hq] = a_a
            a_scr[rows, :, Hq:] = a_b

            def merge_body(b, c):
                for h in range(Hq):
                    cols = slice(h * Dh, (h + 1) * Dh)
                    o_scr[b, :, cols] = (
                        o_scr[b, :, cols] * a_scr[b, :, h:h + 1]
                        + o_recv[stage, b, :, cols].astype(jnp.float32)
                        * a_scr[b, :, Hq + h:Hq + h + 1])
                return c
            lax.fori_loop(2 * half, 2 * half + 2, merge_body, 0)

        def finish(half):
            rows = slice(2 * half, 2 * half + 2)
            a_scr[rows, :, :Hq] = 1.0 / ml_scr[rows, :, Hq:]

            def out_body(b, c):
                for h in range(Hq):
                    cols = slice(h * Dh, (h + 1) * Dh)
                    o_scr[b, :, cols] = o_scr[b, :, cols] * a_scr[b, :, h:h + 1]
                out_ref[b] = lax.dot_general(
                    o_scr[b].astype(jnp.bfloat16), wo_ref[...],
                    (((1,), (0,)), ((), ())), preferred_element_type=jnp.float32)
                return c
            lax.fori_loop(2 * half, 2 * half + 2, out_body, 0)

        lax.fori_loop(0, 2, attn_body, 0)
        cast_half(0)
        ex00_o, ex00_ml = exchange(0, 0, p1)
        ex00_o.start()
        ex00_ml.start()

        lax.fori_loop(2, 4, attn_body, 0)
        cast_half(1)
        ex01_o, ex01_ml = exchange(0, 1, p1)
        ex01_o.start()
        ex01_ml.start()

        ex00_o.wait_recv()
        ex00_ml.wait_recv()
        merge(0, 0)

        ex00_o.wait_send()
        ex00_ml.wait_send()
        cast_half(0)
        ex10_o, ex10_ml = exchange(1, 0, p2)
        ex10_o.start()
        ex10_ml.start()

        ex01_o.wait_recv()
        ex01_ml.wait_recv()
        merge(0, 1)

        ex01_o.wait_send()
        ex01_ml.wait_send()
        cast_half(1)
        ex11_o, ex11_ml = exchange(1, 1, p2)
        ex11_o.start()
        ex11_ml.start()

        ex10_o.wait_recv()
        ex10_ml.wait_recv()
        merge(1, 0)
        finish(0)

        ex11_o.wait_recv()
        ex11_ml.wait_recv()
        merge(1, 1)
        finish(1)

        ex10_o.wait_send()
        ex10_ml.wait_send()
        ex11_o.wait_send()
        ex11_ml.wait_send()

    return pl.pallas_call(
        body,
        out_shape=jax.ShapeDtypeStruct((B, Sq, D), jnp.float32),
        in_specs=[pl.BlockSpec(memory_space=pltpu.VMEM)] * 3
        + [pl.BlockSpec(memory_space=pl.ANY)] * 2,
        out_specs=pl.BlockSpec(memory_space=pltpu.VMEM),
        scratch_shapes=[
            pltpu.VMEM((B, Sq, D), jnp.bfloat16),
            pltpu.VMEM((B, Sq, D), jnp.float32),
            pltpu.VMEM((B, Sq, 2 * Hq), jnp.float32),
            pltpu.VMEM((B, Sq, 2 * Hq), jnp.float32),
            pltpu.VMEM((2, B, Sq, D), jnp.bfloat16),
            pltpu.VMEM((2, B, Sq, 2 * Hq), jnp.float32),
            pltpu.VMEM((2, Skv, Dh), jnp.float32),
            pltpu.VMEM((2, Skv, Dh), jnp.float32),
            pltpu.SemaphoreType.DMA((2, 2)),
            pltpu.SemaphoreType.DMA((8,)),
            pltpu.SemaphoreType.DMA((8,)),
        ],
        compiler_params=pltpu.CompilerParams(
            collective_id=0, vmem_limit_bytes=100 * 1024 * 1024),
    )(xb, wq, wo, K_ext, V_ext)


# device time: 85784 ns/iter; 1.7930x vs baseline; 1.0734x over previous
import jax
import jax.numpy as jnp
from jax import lax
from jax.experimental import pallas as pl
from jax.experimental.pallas import tpu as pltpu

B, Sq, D, Hq, Dh = 4, 256, 1024, 8, 128
NBH = B * Hq
SCALE = 0.08838834764831843


def kernel(x, Wq, Wo, K_ext, V_ext):
    Skv = K_ext.shape[1]

    def body(x_ref, wq_ref, wo_ref, k_ref, v_ref, out_ref,
             q_scr, o_scr, ml_scr, a_scr, o_recv, ml_recv,
             k_buf, v_buf, w_scr, kv_sems, send_sems, recv_sems):
        o_send = q_scr
        my = lax.axis_index("i")
        p1 = my ^ 1
        p2 = 3 - my

        bar = pltpu.get_barrier_semaphore()
        for nbr in (p1, p2):
            pl.semaphore_signal(bar, inc=1, device_id=(nbr,),
                                device_id_type=pl.DeviceIdType.MESH)
        pl.semaphore_wait(bar, 2)

        def kv_copies(i, slot):
            b = i // Hq
            h = i % Hq
            ck = pltpu.make_async_copy(
                k_ref.at[b, :, h, :], k_buf.at[slot], kv_sems.at[slot, 0])
            cv = pltpu.make_async_copy(
                v_ref.at[b, :, h, :], v_buf.at[slot], kv_sems.at[slot, 1])
            return ck, cv

        ck0, cv0 = kv_copies(0, 0)
        ck0.start()
        cv0.start()

        w_scr[...] = wq_ref[...].astype(jnp.bfloat16)

        def attn_body(b, c):
            qb = lax.dot_general(x_ref[b].astype(jnp.bfloat16), w_scr[...],
                                 (((1,), (0,)), ((), ())),
                                 preferred_element_type=jnp.float32)
            q_scr[b] = (qb * SCALE).astype(jnp.bfloat16)
            for h in range(Hq):
                i = b * Hq + h
                slot = h % 2

                @pl.when(i + 1 < NBH)
                def _():
                    ck, cv = kv_copies(i + 1, (h + 1) % 2)
                    ck.start()
                    cv.start()

                ck, cv = kv_copies(i, slot)
                ck.wait()
                cv.wait()
                cols = slice(h * Dh, (h + 1) * Dh)
                qh = q_scr[b, :, cols]
                kh = k_buf[slot].astype(jnp.bfloat16)
                vh = v_buf[slot].astype(jnp.bfloat16)
                s = lax.dot_general(qh, kh, (((1,), (1,)), ((), ())),
                                    preferred_element_type=jnp.float32)
                m = jnp.max(s, axis=1, keepdims=True)
                p = jnp.exp(s - m)
                l = jnp.sum(p, axis=1, keepdims=True)
                o = lax.dot_general(p.astype(jnp.bfloat16), vh,
                                    (((1,), (0,)), ((), ())),
                                    preferred_element_type=jnp.float32)
                o_scr[b, :, cols] = o
                ml_scr[b, :, h:h + 1] = m
                ml_scr[b, :, Hq + h:Hq + h + 1] = l
            return c

        def exchange(stage, half, partner):
            rows = pl.ds(2 * half, 2)
            idx = stage * 4 + half * 2
            r_o = pltpu.make_async_remote_copy(
                src_ref=o_send.at[rows], dst_ref=o_recv.at[stage, rows],
                send_sem=send_sems.at[idx], recv_sem=recv_sems.at[idx],
                device_id=(partner,), device_id_type=pl.DeviceIdType.MESH)
            r_ml = pltpu.make_async_remote_copy(
                src_ref=ml_scr.at[rows], dst_ref=ml_recv.at[stage, rows],
                send_sem=send_sems.at[idx + 1], recv_sem=recv_sems.at[idx + 1],
                device_id=(partner,), device_id_type=pl.DeviceIdType.MESH)
            return r_o, r_ml

        def cast_half(half):
            rows = pl.ds(2 * half, 2)
            o_send[rows] = o_scr[rows].astype(jnp.bfloat16)

        def merge(stage, half):
            rows = slice(2 * half, 2 * half + 2)
            ml_a = ml_scr[rows]
            ml_b = ml_recv[stage, rows]
            m_new = jnp.maximum(ml_a[..., :Hq], ml_b[..., :Hq])
            a_a = jnp.exp(ml_a[..., :Hq] - m_new)
            a_b = jnp.exp(ml_b[..., :Hq] - m_new)
            ml_scr[rows, :, :Hq] = m_new
            ml_scr[rows, :, Hq:] = ml_a[..., Hq:] * a_a + ml_b[..., Hq:] * a_b
            a_scr[rows, :, :Hq] = a_a
            a_scr[rows, :, Hq:] = a_b

            def merge_body(b, c):
                for h in range(Hq):
                    cols = slice(h * Dh, (h + 1) * Dh)
                    o_scr[b, :, cols] = (
                        o_scr[b, :, cols] * a_scr[b, :, h:h + 1]
                        + o_recv[stage, b, :, cols].astype(jnp.float32)
                        * a_scr[b, :, Hq + h:Hq + h + 1])
                return c
            lax.fori_loop(2 * half, 2 * half + 2, merge_body, 0)

        def finish(half):
            rows = slice(2 * half, 2 * half + 2)
            a_scr[rows, :, :Hq] = 1.0 / ml_scr[rows, :, Hq:]

            def out_body(b, c):
                for h in range(Hq):
                    cols = slice(h * Dh, (h + 1) * Dh)
                    q_scr[b, :, cols] = (o_scr[b, :, cols]
                                         * a_scr[b, :, h:h + 1]).astype(jnp.bfloat16)
                out_ref[b] = lax.dot_general(
                    q_scr[b], w_scr[...],
                    (((1,), (0,)), ((), ())), preferred_element_type=jnp.float32)
                return c
            lax.fori_loop(2 * half, 2 * half + 2, out_body, 0)

        lax.fori_loop(0, 2, attn_body, 0)
        cast_half(0)
        ex00_o, ex00_ml = exchange(0, 0, p1)
        ex00_o.start()
        ex00_ml.start()

        lax.fori_loop(2, 4, attn_body, 0)
        cast_half(1)
        ex01_o, ex01_ml = exchange(0, 1, p1)
        ex01_o.start()
        ex01_ml.start()

        ex00_o.wait_recv()
        ex00_ml.wait_recv()
        merge(0, 0)

        ex00_o.wait_send()
        ex00_ml.wait_send()
        cast_half(0)
        ex10_o, ex10_ml = exchange(1, 0, p2)
        ex10_o.start()
        ex10_ml.start()

        ex01_o.wait_recv()
        ex01_ml.wait_recv()
        merge(0, 1)

        ex01_o.wait_send()
        ex01_ml.wait_send()
        cast_half(1)
        ex11_o, ex11_ml = exchange(1, 1, p2)
        ex11_o.start()
        ex11_ml.start()

        w_scr[...] = wo_ref[...].astype(jnp.bfloat16)

        ex10_o.wait_recv()
        ex10_ml.wait_recv()
        merge(1, 0)
        ex10_o.wait_send()
        ex10_ml.wait_send()
        finish(0)

        ex11_o.wait_recv()
        ex11_ml.wait_recv()
        merge(1, 1)
        ex11_o.wait_send()
        ex11_ml.wait_send()
        finish(1)

    return pl.pallas_call(
        body,
        out_shape=jax.ShapeDtypeStruct((B, Sq, D), jnp.float32),
        in_specs=[pl.BlockSpec(memory_space=pltpu.VMEM)] * 3
        + [pl.BlockSpec(memory_space=pl.ANY)] * 2,
        out_specs=pl.BlockSpec(memory_space=pltpu.VMEM),
        scratch_shapes=[
            pltpu.VMEM((B, Sq, D), jnp.bfloat16),
            pltpu.VMEM((B, Sq, D), jnp.float32),
            pltpu.VMEM((B, Sq, 2 * Hq), jnp.float32),
            pltpu.VMEM((B, Sq, 2 * Hq), jnp.float32),
            pltpu.VMEM((2, B, Sq, D), jnp.bfloat16),
            pltpu.VMEM((2, B, Sq, 2 * Hq), jnp.float32),
            pltpu.VMEM((2, Skv, Dh), jnp.float32),
            pltpu.VMEM((2, Skv, Dh), jnp.float32),
            pltpu.VMEM((D, D), jnp.bfloat16),
            pltpu.SemaphoreType.DMA((2, 2)),
            pltpu.SemaphoreType.DMA((8,)),
            pltpu.SemaphoreType.DMA((8,)),
        ],
        compiler_params=pltpu.CompilerParams(
            collective_id=0, vmem_limit_bytes=100 * 1024 * 1024),
    )(x, Wq, Wo, K_ext, V_ext)


# device time: 84917 ns/iter; 1.8113x vs baseline; 1.0102x over previous
import jax
import jax.numpy as jnp
from jax import lax
from jax.experimental import pallas as pl
from jax.experimental.pallas import tpu as pltpu

B, Sq, D, Hq, Dh = 4, 256, 1024, 8, 128
NBH = B * Hq
SCALE = 0.08838834764831843


def kernel(x, Wq, Wo, K_ext, V_ext):
    Skv = K_ext.shape[1]

    def body(x_ref, wq_ref, wo_ref, k_ref, v_ref, out_ref,
             q_scr, o_scr, ml_scr, a_scr, o_recv, ml_recv,
             k_buf, v_buf, w_scr, kv_sems, send_sems, recv_sems):
        o_send = q_scr
        my = lax.axis_index("i")
        p1 = my ^ 1
        p2 = 3 - my

        bar = pltpu.get_barrier_semaphore()
        for nbr in (p1, p2):
            pl.semaphore_signal(bar, inc=1, device_id=(nbr,),
                                device_id_type=pl.DeviceIdType.MESH)
        pl.semaphore_wait(bar, 2)

        def kv_copies(i, slot):
            b = i // Hq
            h = i % Hq
            ck = pltpu.make_async_copy(
                k_ref.at[b, :, h, :], k_buf.at[slot], kv_sems.at[slot, 0])
            cv = pltpu.make_async_copy(
                v_ref.at[b, :, h, :], v_buf.at[slot], kv_sems.at[slot, 1])
            return ck, cv

        ck0, cv0 = kv_copies(0, 0)
        ck0.start()
        cv0.start()

        w_scr[...] = wq_ref[...].astype(jnp.bfloat16)

        def attn_body(b, c):
            qb = lax.dot_general(x_ref[b].astype(jnp.bfloat16), w_scr[...],
                                 (((1,), (0,)), ((), ())),
                                 preferred_element_type=jnp.float32)
            q_scr[b] = (qb * SCALE).astype(jnp.bfloat16)
            for h in range(Hq):
                i = b * Hq + h
                slot = h % 2

                @pl.when(i + 1 < NBH)
                def _():
                    ck, cv = kv_copies(i + 1, (h + 1) % 2)
                    ck.start()
                    cv.start()

                ck, cv = kv_copies(i, slot)
                ck.wait()
                cv.wait()
                cols = slice(h * Dh, (h + 1) * Dh)
                qh = q_scr[b, :, cols]
                kh = k_buf[slot].astype(jnp.bfloat16)
                vh = v_buf[slot].astype(jnp.bfloat16)
                s = lax.dot_general(qh, kh, (((1,), (1,)), ((), ())),
                                    preferred_element_type=jnp.float32)
                m = jnp.max(s, axis=1, keepdims=True)
                p = jnp.exp(s - m)
                l = jnp.sum(p, axis=1, keepdims=True)
                o = lax.dot_general(p.astype(jnp.bfloat16), vh,
                                    (((1,), (0,)), ((), ())),
                                    preferred_element_type=jnp.float32)
                o_scr[b, :, cols] = o
                q_scr[b, :, cols] = o.astype(jnp.bfloat16)
                ml_scr[b, :, h:h + 1] = m
                ml_scr[b, :, Hq + h:Hq + h + 1] = l
            return c

        def exchange(stage, half, partner):
            rows = pl.ds(2 * half, 2)
            idx = stage * 4 + half * 2
            r_o = pltpu.make_async_remote_copy(
                src_ref=o_send.at[rows], dst_ref=o_recv.at[stage, rows],
                send_sem=send_sems.at[idx], recv_sem=recv_sems.at[idx],
                device_id=(partner,), device_id_type=pl.DeviceIdType.MESH)
            r_ml = pltpu.make_async_remote_copy(
                src_ref=ml_scr.at[rows], dst_ref=ml_recv.at[stage, rows],
                send_sem=send_sems.at[idx + 1], recv_sem=recv_sems.at[idx + 1],
                device_id=(partner,), device_id_type=pl.DeviceIdType.MESH)
            return r_o, r_ml

        def merge0(half):
            rows = slice(2 * half, 2 * half + 2)
            ml_a = ml_scr[rows]
            ml_b = ml_recv[0, rows]
            m_new = jnp.maximum(ml_a[..., :Hq], ml_b[..., :Hq])
            a_a = jnp.exp(ml_a[..., :Hq] - m_new)
            a_b = jnp.exp(ml_b[..., :Hq] - m_new)
            ml_scr[rows, :, :Hq] = m_new
            ml_scr[rows, :, Hq:] = ml_a[..., Hq:] * a_a + ml_b[..., Hq:] * a_b
            a_scr[rows, :, :Hq] = a_a
            a_scr[rows, :, Hq:] = a_b

            def merge_body(b, c):
                for h in range(Hq):
                    cols = slice(h * Dh, (h + 1) * Dh)
                    v = (o_scr[b, :, cols] * a_scr[b, :, h:h + 1]
                         + o_recv[0, b, :, cols].astype(jnp.float32)
                         * a_scr[b, :, Hq + h:Hq + h + 1])
                    o_scr[b, :, cols] = v
                    q_scr[b, :, cols] = v.astype(jnp.bfloat16)
                return c
            lax.fori_loop(2 * half, 2 * half + 2, merge_body, 0)

        def merge_finish(half):
            rows = slice(2 * half, 2 * half + 2)
            ml_a = ml_scr[rows]
            ml_b = ml_recv[1, rows]
            m_new = jnp.maximum(ml_a[..., :Hq], ml_b[..., :Hq])
            a_a = jnp.exp(ml_a[..., :Hq] - m_new)
            a_b = jnp.exp(ml_b[..., :Hq] - m_new)
            l_new = ml_a[..., Hq:] * a_a + ml_b[..., Hq:] * a_b
            a_scr[rows, :, :Hq] = a_a / l_new
            a_scr[rows, :, Hq:] = a_b / l_new

            def out_body(b, c):
                for h in range(Hq):
                    cols = slice(h * Dh, (h + 1) * Dh)
                    q_scr[b, :, cols] = (
                        o_scr[b, :, cols] * a_scr[b, :, h:h + 1]
                        + o_recv[1, b, :, cols].astype(jnp.float32)
                        * a_scr[b, :, Hq + h:Hq + h + 1]).astype(jnp.bfloat16)
                out_ref[b] = lax.dot_general(
                    q_scr[b], w_scr[...],
                    (((1,), (0,)), ((), ())), preferred_element_type=jnp.float32)
                return c
            lax.fori_loop(2 * half, 2 * half + 2, out_body, 0)

        lax.fori_loop(0, 2, attn_body, 0)
        ex00_o, ex00_ml = exchange(0, 0, p1)
        ex00_o.start()
        ex00_ml.start()

        lax.fori_loop(2, 4, attn_body, 0)
        ex01_o, ex01_ml = exchange(0, 1, p1)
        ex01_o.start()
        ex01_ml.start()

        ex00_o.wait_recv()
        ex00_ml.wait_recv()
        ex00_o.wait_send()
        ex00_ml.wait_send()
        merge0(0)
        ex10_o, ex10_ml = exchange(1, 0, p2)
        ex10_o.start()
        ex10_ml.start()

        ex01_o.wait_recv()
        ex01_ml.wait_recv()
        ex01_o.wait_send()
        ex01_ml.wait_send()
        merge0(1)
        ex11_o, ex11_ml = exchange(1, 1, p2)
        ex11_o.start()
        ex11_ml.start()

        w_scr[...] = wo_ref[...].astype(jnp.bfloat16)

        ex10_o.wait_recv()
        ex10_ml.wait_recv()
        ex10_o.wait_send()
        ex10_ml.wait_send()
        merge_finish(0)

        ex11_o.wait_recv()
        ex11_ml.wait_recv()
        ex11_o.wait_send()
        ex11_ml.wait_send()
        merge_finish(1)

    return pl.pallas_call(
        body,
        out_shape=jax.ShapeDtypeStruct((B, Sq, D), jnp.float32),
        in_specs=[pl.BlockSpec(memory_space=pltpu.VMEM)] * 3
        + [pl.BlockSpec(memory_space=pl.ANY)] * 2,
        out_specs=pl.BlockSpec(memory_space=pltpu.VMEM),
        scratch_shapes=[
            pltpu.VMEM((B, Sq, D), jnp.bfloat16),
            pltpu.VMEM((B, Sq, D), jnp.float32),
            pltpu.VMEM((B, Sq, 2 * Hq), jnp.float32),
            pltpu.VMEM((B, Sq, 2 * Hq), jnp.float32),
            pltpu.VMEM((2, B, Sq, D), jnp.bfloat16),
            pltpu.VMEM((2, B, Sq, 2 * Hq), jnp.float32),
            pltpu.VMEM((2, Skv, Dh), jnp.float32),
            pltpu.VMEM((2, Skv, Dh), jnp.float32),
            pltpu.VMEM((D, D), jnp.bfloat16),
            pltpu.SemaphoreType.DMA((2, 2)),
            pltpu.SemaphoreType.DMA((8,)),
            pltpu.SemaphoreType.DMA((8,)),
        ],
        compiler_params=pltpu.CompilerParams(
            collective_id=0, vmem_limit_bytes=100 * 1024 * 1024),
    )(x, Wq, Wo, K_ext, V_ext)


# device time: 74792 ns/iter; 2.0565x vs baseline; 1.1354x over previous
import jax
import jax.numpy as jnp
from jax import lax
from jax.experimental import pallas as pl
from jax.experimental.pallas import tpu as pltpu

B, Sq, D, Hq, Dh = 4, 256, 1024, 8, 128
NBH = B * Hq
SCALE = 0.08838834764831843
NSLOT = 4
PF = NSLOT - 1


def kernel(x, Wq, Wo, K_ext, V_ext):
    Skv = K_ext.shape[1]

    def body(x_ref, wq_ref, wo_ref, k_ref, v_ref, out_ref,
             q_scr, o_scr, ml_scr, a_scr, o_recv, ml_recv,
             k_buf, v_buf, w_scr, kv_sems, send_sems, recv_sems):
        o_send = q_scr
        my = lax.axis_index("i")
        p1 = my ^ 1
        p2 = 3 - my

        bar = pltpu.get_barrier_semaphore()
        for nbr in (p1, p2):
            pl.semaphore_signal(bar, inc=1, device_id=(nbr,),
                                device_id_type=pl.DeviceIdType.MESH)
        pl.semaphore_wait(bar, 2)

        def kv_copies(i, slot):
            b = i // Hq
            h = i % Hq
            ck = pltpu.make_async_copy(
                k_ref.at[b, :, h, :], k_buf.at[slot], kv_sems.at[slot, 0])
            cv = pltpu.make_async_copy(
                v_ref.at[b, :, h, :], v_buf.at[slot], kv_sems.at[slot, 1])
            return ck, cv

        for i0 in range(PF):
            ck, cv = kv_copies(i0, i0)
            ck.start()
            cv.start()

        w_scr[...] = wq_ref[...].astype(jnp.bfloat16)

        def attn_body(b, c):
            qb = lax.dot_general(x_ref[b].astype(jnp.bfloat16), w_scr[...],
                                 (((1,), (0,)), ((), ())),
                                 preferred_element_type=jnp.float32)
            q_scr[b] = (qb * SCALE).astype(jnp.bfloat16)
            for h in range(Hq):
                i = b * Hq + h
                slot = h % NSLOT

                @pl.when(i + PF < NBH)
                def _():
                    ck, cv = kv_copies(i + PF, (h + PF) % NSLOT)
                    ck.start()
                    cv.start()

                ck, cv = kv_copies(i, slot)
                ck.wait()
                cv.wait()
                cols = slice(h * Dh, (h + 1) * Dh)
                qh = q_scr[b, :, cols]
                kh = k_buf[slot].astype(jnp.bfloat16)
                vh = v_buf[slot].astype(jnp.bfloat16)
                s = lax.dot_general(qh, kh, (((1,), (1,)), ((), ())),
                                    preferred_element_type=jnp.float32)
                m = jnp.max(s, axis=1, keepdims=True)
                p = jnp.exp(s - m)
                l = jnp.sum(p, axis=1, keepdims=True)
                o = lax.dot_general(p.astype(jnp.bfloat16), vh,
                                    (((1,), (0,)), ((), ())),
                                    preferred_element_type=jnp.float32)
                o_scr[b, :, cols] = o
                q_scr[b, :, cols] = o.astype(jnp.bfloat16)
                ml_scr[b, :, h:h + 1] = m
                ml_scr[b, :, Hq + h:Hq + h + 1] = l
            return c

        def exchange(stage, b, partner):
            rows = pl.ds(b, 1)
            idx = stage * 2 * B + b * 2
            r_o = pltpu.make_async_remote_copy(
                src_ref=o_send.at[rows], dst_ref=o_recv.at[stage, rows],
                send_sem=send_sems.at[idx], recv_sem=recv_sems.at[idx],
                device_id=(partner,), device_id_type=pl.DeviceIdType.MESH)
            r_ml = pltpu.make_async_remote_copy(
                src_ref=ml_scr.at[rows], dst_ref=ml_recv.at[stage, rows],
                send_sem=send_sems.at[idx + 1], recv_sem=recv_sems.at[idx + 1],
                device_id=(partner,), device_id_type=pl.DeviceIdType.MESH)
            return r_o, r_ml

        def merge0(b):
            ml_a = ml_scr[b]
            ml_b = ml_recv[0, b]
            m_new = jnp.maximum(ml_a[:, :Hq], ml_b[:, :Hq])
            a_a = jnp.exp(ml_a[:, :Hq] - m_new)
            a_b = jnp.exp(ml_b[:, :Hq] - m_new)
            ml_scr[b, :, :Hq] = m_new
            ml_scr[b, :, Hq:] = ml_a[:, Hq:] * a_a + ml_b[:, Hq:] * a_b
            a_scr[b, :, :Hq] = a_a
            a_scr[b, :, Hq:] = a_b
            for h in range(Hq):
                cols = slice(h * Dh, (h + 1) * Dh)
                v = (o_scr[b, :, cols] * a_scr[b, :, h:h + 1]
                     + o_recv[0, b, :, cols].astype(jnp.float32)
                     * a_scr[b, :, Hq + h:Hq + h + 1])
                o_scr[b, :, cols] = v
                q_scr[b, :, cols] = v.astype(jnp.bfloat16)

        def merge_finish(b):
            ml_a = ml_scr[b]
            ml_b = ml_recv[1, b]
            m_new = jnp.maximum(ml_a[:, :Hq], ml_b[:, :Hq])
            a_a = jnp.exp(ml_a[:, :Hq] - m_new)
            a_b = jnp.exp(ml_b[:, :Hq] - m_new)
            l_new = ml_a[:, Hq:] * a_a + ml_b[:, Hq:] * a_b
            a_scr[b, :, :Hq] = a_a / l_new
            a_scr[b, :, Hq:] = a_b / l_new
            for h in range(Hq):
                cols = slice(h * Dh, (h + 1) * Dh)
                q_scr[b, :, cols] = (
                    o_scr[b, :, cols] * a_scr[b, :, h:h + 1]
                    + o_recv[1, b, :, cols].astype(jnp.float32)
                    * a_scr[b, :, Hq + h:Hq + h + 1]).astype(jnp.bfloat16)
            out_ref[b] = lax.dot_general(
                q_scr[b], w_scr[...],
                (((1,), (0,)), ((), ())), preferred_element_type=jnp.float32)

        ex0 = []
        for b in range(B):
            lax.fori_loop(b, b + 1, attn_body, 0)
            r_o, r_ml = exchange(0, b, p1)
            r_o.start()
            r_ml.start()
            ex0.append((r_o, r_ml))

        w_scr[...] = wo_ref[...].astype(jnp.bfloat16)

        ex1 = []
        for b in range(B):
            r_o, r_ml = ex0[b]
            r_o.wait_recv()
            r_ml.wait_recv()
            r_o.wait_send()
            r_ml.wait_send()
            merge0(b)
            s_o, s_ml = exchange(1, b, p2)
            s_o.start()
            s_ml.start()
            ex1.append((s_o, s_ml))

        for b in range(B):
            r_o, r_ml = ex1[b]
            r_o.wait_recv()
            r_ml.wait_recv()
            r_o.wait_send()
            r_ml.wait_send()
            merge_finish(b)

    return pl.pallas_call(
        body,
        out_shape=jax.ShapeDtypeStruct((B, Sq, D), jnp.float32),
        in_specs=[pl.BlockSpec(memory_space=pltpu.VMEM)] * 3
        + [pl.BlockSpec(memory_space=pl.ANY)] * 2,
        out_specs=pl.BlockSpec(memory_space=pltpu.VMEM),
        scratch_shapes=[
            pltpu.VMEM((B, Sq, D), jnp.bfloat16),
            pltpu.VMEM((B, Sq, D), jnp.float32),
            pltpu.VMEM((B, Sq, 2 * Hq), jnp.float32),
            pltpu.VMEM((B, Sq, 2 * Hq), jnp.float32),
            pltpu.VMEM((2, B, Sq, D), jnp.bfloat16),
            pltpu.VMEM((2, B, Sq, 2 * Hq), jnp.float32),
            pltpu.VMEM((NSLOT, Skv, Dh), jnp.float32),
            pltpu.VMEM((NSLOT, Skv, Dh), jnp.float32),
            pltpu.VMEM((D, D), jnp.bfloat16),
            pltpu.SemaphoreType.DMA((NSLOT, 2)),
            pltpu.SemaphoreType.DMA((16,)),
            pltpu.SemaphoreType.DMA((16,)),
        ],
        compiler_params=pltpu.CompilerParams(
            collective_id=0, vmem_limit_bytes=100 * 1024 * 1024),
    )(x, Wq, Wo, K_ext, V_ext)


# device time: 66079 ns/iter; 2.3277x vs baseline; 1.1319x over previous
import jax
import jax.numpy as jnp
from jax import lax
from jax.experimental import pallas as pl
from jax.experimental.pallas import tpu as pltpu

B, Sq, D, Hq, Dh = 4, 256, 1024, 8, 128
NBH = B * Hq
SCALE = 0.08838834764831843
NSLOT = 4
PF = NSLOT - 1


def kernel(x, Wq, Wo, K_ext, V_ext):
    Skv = K_ext.shape[1]

    def body(x_ref, wq_ref, wo_ref, k_ref, v_ref, out_ref,
             q_scr, o_scr, ml_scr, a_scr, o_recv, ml_recv,
             k_buf, v_buf, w_scr, kv_sems, send_sems, recv_sems):
        o_send = q_scr
        my = lax.axis_index("i")
        p1 = my ^ 1
        p2 = 3 - my

        bar = pltpu.get_barrier_semaphore()
        for nbr in (p1, p2):
            pl.semaphore_signal(bar, inc=1, device_id=(nbr,),
                                device_id_type=pl.DeviceIdType.MESH)
        pl.semaphore_wait(bar, 2)

        def kv_copies(i, slot):
            b = i // Hq
            h = i % Hq
            ck = pltpu.make_async_copy(
                k_ref.at[b, :, h, :], k_buf.at[slot], kv_sems.at[slot, 0])
            cv = pltpu.make_async_copy(
                v_ref.at[b, :, h, :], v_buf.at[slot], kv_sems.at[slot, 1])
            return ck, cv

        for i0 in range(PF):
            ck, cv = kv_copies(i0, i0)
            ck.start()
            cv.start()

        w_scr[...] = wq_ref[...].astype(jnp.bfloat16)

        def attn_body(b, c):
            qb = lax.dot_general(x_ref[b].astype(jnp.bfloat16), w_scr[...],
                                 (((1,), (0,)), ((), ())),
                                 preferred_element_type=jnp.float32)
            q_scr[b] = (qb * SCALE).astype(jnp.bfloat16)
            for h in range(Hq):
                i = b * Hq + h
                slot = h % NSLOT

                @pl.when(i + PF < NBH)
                def _():
                    ck, cv = kv_copies(i + PF, (h + PF) % NSLOT)
                    ck.start()
                    cv.start()

                ck, cv = kv_copies(i, slot)
                ck.wait()
                cv.wait()
                cols = slice(h * Dh, (h + 1) * Dh)
                qh = q_scr[b, :, cols]
                kh = k_buf[slot].astype(jnp.bfloat16)
                vh = v_buf[slot].astype(jnp.bfloat16)
                s = lax.dot_general(qh, kh, (((1,), (1,)), ((), ())),
                                    preferred_element_type=jnp.float32)
                m = jnp.max(s, axis=1, keepdims=True)
                p = jnp.exp(s - m)
                l = jnp.sum(p, axis=1, keepdims=True)
                o = lax.dot_general(p.astype(jnp.bfloat16), vh,
                                    (((1,), (0,)), ((), ())),
                                    preferred_element_type=jnp.float32)
                o_scr[b, :, cols] = o
                q_scr[b, :, cols] = o.astype(jnp.bfloat16)
                ml_scr[b, :, h:h + 1] = m
                ml_scr[b, :, Hq + h:Hq + h + 1] = l
            return c

        def exchange(stage, b, partner):
            rows = pl.ds(b, 1)
            idx = stage * 2 * B + b * 2
            r_o = pltpu.make_async_remote_copy(
                src_ref=o_send.at[rows], dst_ref=o_recv.at[stage, rows],
                send_sem=send_sems.at[idx], recv_sem=recv_sems.at[idx],
                device_id=(partner,), device_id_type=pl.DeviceIdType.MESH)
            r_ml = pltpu.make_async_remote_copy(
                src_ref=ml_scr.at[rows], dst_ref=ml_recv.at[stage, rows],
                send_sem=send_sems.at[idx + 1], recv_sem=recv_sems.at[idx + 1],
                device_id=(partner,), device_id_type=pl.DeviceIdType.MESH)
            return r_o, r_ml

        def merge0(b):
            ml_a = ml_scr[b]
            ml_b = ml_recv[0, b]
            m_new = jnp.maximum(ml_a[:, :Hq], ml_b[:, :Hq])
            a_a = jnp.exp(ml_a[:, :Hq] - m_new)
            a_b = jnp.exp(ml_b[:, :Hq] - m_new)
            ml_scr[b, :, :Hq] = m_new
            ml_scr[b, :, Hq:] = ml_a[:, Hq:] * a_a + ml_b[:, Hq:] * a_b
            a_scr[b, :, :Hq] = a_a
            a_scr[b, :, Hq:] = a_b
            for h in range(Hq):
                cols = slice(h * Dh, (h + 1) * Dh)
                v = (o_scr[b, :, cols] * a_scr[b, :, h:h + 1]
                     + o_recv[0, b, :, cols].astype(jnp.float32)
                     * a_scr[b, :, Hq + h:Hq + h + 1])
                o_scr[b, :, cols] = v
                q_scr[b, :, cols] = v.astype(jnp.bfloat16)

        def merge_finish(b):
            ml_a = ml_scr[b]
            ml_b = ml_recv[1, b]
            m_new = jnp.maximum(ml_a[:, :Hq], ml_b[:, :Hq])
            a_a = jnp.exp(ml_a[:, :Hq] - m_new)
            a_b = jnp.exp(ml_b[:, :Hq] - m_new)
            l_new = ml_a[:, Hq:] * a_a + ml_b[:, Hq:] * a_b
            a_scr[b, :, :Hq] = a_a / l_new
            a_scr[b, :, Hq:] = a_b / l_new
            for h in range(Hq):
                cols = slice(h * Dh, (h + 1) * Dh)
                q_scr[b, :, cols] = (
                    o_scr[b, :, cols] * a_scr[b, :, h:h + 1]
                    + o_recv[1, b, :, cols].astype(jnp.float32)
                    * a_scr[b, :, Hq + h:Hq + h + 1]).astype(jnp.bfloat16)
            out_ref[b] = lax.dot_general(
                q_scr[b], w_scr[...],
                (((1,), (0,)), ((), ())), preferred_element_type=jnp.float32)

        ex0 = []
        ex1 = []

        def merge_and_send1(b):
            r_o, r_ml = ex0[b]
            r_o.wait_recv()
            r_ml.wait_recv()
            r_o.wait_send()
            r_ml.wait_send()
            merge0(b)
            s_o, s_ml = exchange(1, b, p2)
            s_o.start()
            s_ml.start()
            ex1.append((s_o, s_ml))

        for b in range(B):
            lax.fori_loop(b, b + 1, attn_body, 0)
            r_o, r_ml = exchange(0, b, p1)
            r_o.start()
            r_ml.start()
            ex0.append((r_o, r_ml))
            if b >= 1:
                merge_and_send1(b - 1)
        merge_and_send1(B - 1)

        w_scr[...] = wo_ref[...].astype(jnp.bfloat16)

        for b in range(B):
            r_o, r_ml = ex1[b]
            r_o.wait_recv()
            r_ml.wait_recv()
            r_o.wait_send()
            r_ml.wait_send()
            merge_finish(b)

    return pl.pallas_call(
        body,
        out_shape=jax.ShapeDtypeStruct((B, Sq, D), jnp.float32),
        in_specs=[pl.BlockSpec(memory_space=pltpu.VMEM)] * 3
        + [pl.BlockSpec(memory_space=pl.ANY)] * 2,
        out_specs=pl.BlockSpec(memory_space=pltpu.VMEM),
        scratch_shapes=[
            pltpu.VMEM((B, Sq, D), jnp.bfloat16),
            pltpu.VMEM((B, Sq, D), jnp.float32),
            pltpu.VMEM((B, Sq, 2 * Hq), jnp.float32),
            pltpu.VMEM((B, Sq, 2 * Hq), jnp.float32),
            pltpu.VMEM((2, B, Sq, D), jnp.bfloat16),
            pltpu.VMEM((2, B, Sq, 2 * Hq), jnp.float32),
            pltpu.VMEM((NSLOT, Skv, Dh), jnp.float32),
            pltpu.VMEM((NSLOT, Skv, Dh), jnp.float32),
            pltpu.VMEM((D, D), jnp.bfloat16),
            pltpu.SemaphoreType.DMA((NSLOT, 2)),
            pltpu.SemaphoreType.DMA((16,)),
            pltpu.SemaphoreType.DMA((16,)),
        ],
        compiler_params=pltpu.CompilerParams(
            collective_id=0, vmem_limit_bytes=100 * 1024 * 1024),
    )(x, Wq, Wo, K_ext, V_ext)
